# baseline (speedup 1.0000x reference)
import jax
import jax.numpy as jnp
from jax.experimental import pallas as pl
from jax.experimental.pallas import tpu as pltpu

IN_SIZE = 4096
OUT_SIZE = 1024
TB = 1024


def _stagger_body(
    x_ref, u_ref, w1_ref, b1_ref, wh_ref, bh_ref, wo_ref, out_ref, s0, s1
):
    i = pl.program_id(0)
    nsteps = pl.num_programs(0)

    def compute_hm():
        x = x_ref[...].astype(jnp.bfloat16)
        y1 = jnp.tanh(
            jnp.dot(
                x,
                w1_ref[...].astype(jnp.bfloat16),
                preferred_element_type=jnp.float32,
            )
            - b1_ref[...]
        )
        h = jax.nn.sigmoid(
            jnp.dot(y1, wh_ref[...], preferred_element_type=jnp.float32)
            - bh_ref[...]
        )
        u = u_ref[...]
        col = jax.lax.broadcasted_iota(jnp.int32, (1, 64), 1)
        return jnp.where((col // 16) == u, h, 0.0) + ((col - 48) == u).astype(
            jnp.float32
        )

    def write_out(s):
        out_ref[...] = jnp.dot(
            s[...], wo_ref[...], preferred_element_type=jnp.float32
        )

    even = i % 2 == 0

    @pl.when(jnp.logical_and(i < nsteps - 1, even))
    def _():
        s0[...] = compute_hm()

    @pl.when(jnp.logical_and(i < nsteps - 1, jnp.logical_not(even)))
    def _():
        s1[...] = compute_hm()

    @pl.when(jnp.logical_and(i > 0, even))
    def _():
        write_out(s1)

    @pl.when(jnp.logical_and(i > 0, jnp.logical_not(even)))
    def _():
        write_out(s0)


def kernel(x, u, w1, b1, w2, b2, w3, b3, w4, b4, w5, b5, w6, b6, w7, b7):
    x = x.astype(jnp.float32)
    B = x.shape[0]
    wh = jnp.zeros((8, 64), jnp.float32)
    wh = wh.at[:, 0:16].set(w2).at[:, 16:32].set(w4).at[:, 32:48].set(w6)
    bh = jnp.zeros((1, 64), jnp.float32)
    bh = bh.at[0, 0:16].set(b2).at[0, 16:32].set(b4).at[0, 32:48].set(b6)
    wo = jnp.zeros((64, OUT_SIZE), jnp.float32)
    wo = wo.at[0:16, :].set(w3).at[16:32, :].set(w5).at[32:48, :].set(w7)
    wo = wo.at[48, :].set(-b3).at[49, :].set(-b5).at[50, :].set(-b7)

    nb = B // TB
    return pl.pallas_call(
        _stagger_body,
        grid=(nb + 1,),
        in_specs=[
            pl.BlockSpec((TB, IN_SIZE), lambda i: (jnp.minimum(i, nb - 1), 0)),
            pl.BlockSpec((TB, 1), lambda i: (jnp.minimum(i, nb - 1), 0)),
            pl.BlockSpec((IN_SIZE, 8), lambda i: (0, 0)),
            pl.BlockSpec((1, 8), lambda i: (0, 0)),
            pl.BlockSpec((8, 64), lambda i: (0, 0)),
            pl.BlockSpec((1, 64), lambda i: (0, 0)),
            pl.BlockSpec((64, OUT_SIZE), lambda i: (0, 0)),
        ],
        out_specs=pl.BlockSpec(
            (TB, OUT_SIZE), lambda i: (jnp.maximum(i - 1, 0), 0)
        ),
        out_shape=jax.ShapeDtypeStruct((B, OUT_SIZE), jnp.float32),
        scratch_shapes=[
            pltpu.VMEM((TB, 64), jnp.float32),
            pltpu.VMEM((TB, 64), jnp.float32),
        ],
        compiler_params=pltpu.CompilerParams(
            dimension_semantics=("arbitrary",)
        ),
    )(x, u.reshape(B, 1), w1, b1.reshape(1, 8), wh, bh, wo)
